# Initial kernel scaffold; baseline (speedup 1.0000x reference)
import jax
import jax.numpy as jnp
from jax.experimental import pallas as pl


def kernel(qsa_p, pb, edge_out_s, seg_s, reward, done):
    raise NotImplementedError("write your pallas kernel here")


# speedup vs baseline: 16.2329x; 16.2329x over previous
import functools

import jax
import jax.numpy as jnp
from jax import lax
from jax.experimental import pallas as pl
from jax.experimental.pallas import tpu as pltpu
from jax.experimental.pallas import tpu_sc as plsc

_N = 3_200_000
_T = 100_000
_PAD_T = 100_352
_SLICE = _PAD_T // 16
_C = 1600
_PER_TILE = _N // 16
_CHUNKS = _PER_TILE // _C
_LOG_REG_C = 2.5e-5
_LEAF_COEF = 10.0
_CLIP = 10.0


def _sc_segment_sums(qsa_p, pb, edge_out_s, seg_s):
  mesh = plsc.VectorSubcoreMesh(core_axis_name="c", subcore_axis_name="s")

  @functools.partial(
      pl.kernel,
      out_type=jax.ShapeDtypeStruct((2 * _PAD_T,), jnp.float32),
      mesh=mesh,
      scratch_types=[
          pltpu.VMEM((_C,), jnp.float32),
          pltpu.VMEM((_C,), jnp.int32),
          pltpu.VMEM((_SLICE,), jnp.float32),
          pltpu.VMEM_SHARED((_PAD_T,), jnp.float32),
      ],
  )
  def k(qsa_hbm, pb_hbm, eos_hbm, seg_hbm, out_hbm, vbuf, ibuf, zbuf, acc):
    c = lax.axis_index("c")
    s = lax.axis_index("s")

    def zero_body(i, carry):
      zbuf[pl.ds(i * 16, 16)] = jnp.zeros((16,), jnp.float32)
      return carry

    lax.fori_loop(0, _SLICE // 16, zero_body, 0)
    pltpu.sync_copy(zbuf, acc.at[pl.ds(s * _SLICE, _SLICE)])
    plsc.subcore_barrier()

    def process(vals_hbm, idx_hbm):
      base = s * _PER_TILE

      def chunk_body(i, carry):
        off = base + i * _C
        pltpu.sync_copy(vals_hbm.at[pl.ds(off, _C)], vbuf)
        pltpu.sync_copy(idx_hbm.at[pl.ds(off, _C)], ibuf)

        def exp_body(j, inner):
          vbuf[pl.ds(j * 16, 16)] = jnp.exp(vbuf[pl.ds(j * 16, 16)])
          return inner

        lax.fori_loop(0, _C // 16, exp_body, 0)
        pltpu.sync_copy(vbuf, acc.at[ibuf], add=True)
        return carry

      lax.fori_loop(0, _CHUNKS, chunk_body, 0)

    @pl.when(c == 0)
    def _():
      process(qsa_hbm, pb_hbm)

    @pl.when(c == 1)
    def _():
      process(eos_hbm, seg_hbm)

    plsc.subcore_barrier()

    pltpu.sync_copy(acc.at[pl.ds(s * _SLICE, _SLICE)], zbuf)

    @pl.when(c == 0)
    def _():
      pltpu.sync_copy(zbuf, out_hbm.at[pl.ds(s * _SLICE, _SLICE)])

    @pl.when(c == 1)
    def _():
      pltpu.sync_copy(zbuf, out_hbm.at[pl.ds(_PAD_T + s * _SLICE, _SLICE)])

  return k(qsa_p, pb, edge_out_s, seg_s)


def _tc_loss_body(acc_ref, rw_ref, dn_ref, out_ref):
  ei = acc_ref[0]
  eo = acc_ref[1]
  rwv = rw_ref[...]
  dnv = dn_ref[...]
  done_b = (dnv > 0.5).astype(jnp.float32)
  inflow = jnp.log(ei + _LOG_REG_C)
  opr = jnp.log(_LOG_REG_C + rwv + eo * (1.0 - done_b))
  l = (inflow - opr) ** 2
  l = jnp.minimum(l, _CLIP)
  rows, cols = rw_ref.shape
  li = (lax.broadcasted_iota(jnp.int32, (rows, cols), 0) * cols
        + lax.broadcasted_iota(jnp.int32, (rows, cols), 1))
  valid = (li < _T).astype(jnp.float32)
  term_num = jnp.sum(l * done_b)
  term_den = jnp.sum(done_b)
  flow_num = jnp.sum(l * (1.0 - done_b) * valid)
  flow_den = jnp.sum((1.0 - done_b) * valid)
  out_ref[0, 0] = (term_num / (term_den + 1e-20) * _LEAF_COEF
                   + flow_num / (flow_den + 1e-20))


def _tc_loss(acc, reward, done, interpret=False):
  acc3 = acc.reshape(2, _PAD_T // 128, 128)
  pad = _PAD_T - _T
  rw = jnp.pad(reward, (0, pad)).reshape(_PAD_T // 128, 128)
  dn = jnp.pad(done, (0, pad)).reshape(_PAD_T // 128, 128)
  out = pl.pallas_call(
      _tc_loss_body,
      out_shape=jax.ShapeDtypeStruct((1, 1), jnp.float32),
      out_specs=pl.BlockSpec(memory_space=pltpu.MemorySpace.SMEM),
      interpret=interpret,
  )(acc3, rw, dn)
  return out.reshape(())


def kernel(qsa_p, pb, edge_out_s, seg_s, reward, done):
  acc = _sc_segment_sums(qsa_p, pb.astype(jnp.int32),
                         edge_out_s, seg_s.astype(jnp.int32))
  return _tc_loss(acc, reward, done)


# speedup vs baseline: 40.5246x; 2.4965x over previous
import functools

import jax
import jax.numpy as jnp
from jax import lax
from jax.experimental import pallas as pl
from jax.experimental.pallas import tpu as pltpu
from jax.experimental.pallas import tpu_sc as plsc

_N = 3_200_000
_T = 100_000
_PAD_T = 100_352
_SLICE = _PAD_T // 16
_C = 4000
_PER_TILE = _N // 16
_CHUNKS = _PER_TILE // _C
_LOG_REG_C = 2.5e-5
_LEAF_COEF = 10.0
_CLIP = 10.0


def _sc_segment_sums(qsa_p, pb, edge_out_s, seg_s):
  mesh = plsc.VectorSubcoreMesh(core_axis_name="c", subcore_axis_name="s")

  @functools.partial(
      pl.kernel,
      out_type=jax.ShapeDtypeStruct((2 * _PAD_T,), jnp.float32),
      mesh=mesh,
      scratch_types=[
          pltpu.VMEM((_C,), jnp.float32),
          pltpu.VMEM((_C,), jnp.float32),
          pltpu.VMEM((_C,), jnp.int32),
          pltpu.VMEM((_C,), jnp.int32),
          pltpu.VMEM((_SLICE,), jnp.float32),
          pltpu.VMEM_SHARED((_PAD_T,), jnp.float32),
          pltpu.SemaphoreType.DMA,
          pltpu.SemaphoreType.DMA,
      ],
  )
  def k(qsa_hbm, pb_hbm, eos_hbm, seg_hbm, out_hbm,
        v0, v1, i0, i1, zbuf, acc, sem0, sem1):
    c = lax.axis_index("c")
    s = lax.axis_index("s")

    def zero_body(i, carry):
      zbuf[pl.ds(i * 16, 16)] = jnp.zeros((16,), jnp.float32)
      return carry

    lax.fori_loop(0, _SLICE // 16, zero_body, 0)
    pltpu.sync_copy(zbuf, acc.at[pl.ds(s * _SLICE, _SLICE)])
    plsc.subcore_barrier()

    def process(vals_hbm, idx_hbm):
      base = s * _PER_TILE

      def fetch(ci, vslot, islot, sem):
        off = base + ci * _C
        pltpu.async_copy(vals_hbm.at[pl.ds(off, _C)], vslot, sem)
        pltpu.async_copy(idx_hbm.at[pl.ds(off, _C)], islot, sem)

      def wait_fetch(vslot, islot, sem):
        pltpu.make_async_copy(vals_hbm.at[pl.ds(0, _C)], vslot, sem).wait()
        pltpu.make_async_copy(idx_hbm.at[pl.ds(0, _C)], islot, sem).wait()

      def do_exp(buf):
        for j in range(_C // 16):
          buf[pl.ds(j * 16, 16)] = jnp.exp(buf[pl.ds(j * 16, 16)])

      fetch(0, v0, i0, sem0)
      fetch(1, v1, i1, sem1)

      def pair_body(k2, carry):
        wait_fetch(v0, i0, sem0)
        do_exp(v0)
        pltpu.sync_copy(v0, acc.at[i0], add=True)

        @pl.when(2 * k2 + 2 < _CHUNKS)
        def _():
          fetch(2 * k2 + 2, v0, i0, sem0)

        wait_fetch(v1, i1, sem1)
        do_exp(v1)
        pltpu.sync_copy(v1, acc.at[i1], add=True)

        @pl.when(2 * k2 + 3 < _CHUNKS)
        def _():
          fetch(2 * k2 + 3, v1, i1, sem1)

        return carry

      lax.fori_loop(0, _CHUNKS // 2, pair_body, 0)

    @pl.when(c == 0)
    def _():
      process(qsa_hbm, pb_hbm)

    @pl.when(c == 1)
    def _():
      process(eos_hbm, seg_hbm)

    plsc.subcore_barrier()

    pltpu.sync_copy(acc.at[pl.ds(s * _SLICE, _SLICE)], zbuf)

    @pl.when(c == 0)
    def _():
      pltpu.sync_copy(zbuf, out_hbm.at[pl.ds(s * _SLICE, _SLICE)])

    @pl.when(c == 1)
    def _():
      pltpu.sync_copy(zbuf, out_hbm.at[pl.ds(_PAD_T + s * _SLICE, _SLICE)])

  return k(qsa_p, pb, edge_out_s, seg_s)


def _tc_loss_body(acc_ref, rw_ref, dn_ref, out_ref):
  ei = acc_ref[0]
  eo = acc_ref[1]
  rwv = rw_ref[...]
  dnv = dn_ref[...]
  done_b = (dnv > 0.5).astype(jnp.float32)
  inflow = jnp.log(ei + _LOG_REG_C)
  opr = jnp.log(_LOG_REG_C + rwv + eo * (1.0 - done_b))
  l = (inflow - opr) ** 2
  l = jnp.minimum(l, _CLIP)
  rows, cols = rw_ref.shape
  li = (lax.broadcasted_iota(jnp.int32, (rows, cols), 0) * cols
        + lax.broadcasted_iota(jnp.int32, (rows, cols), 1))
  valid = (li < _T).astype(jnp.float32)
  term_num = jnp.sum(l * done_b)
  term_den = jnp.sum(done_b)
  flow_num = jnp.sum(l * (1.0 - done_b) * valid)
  flow_den = jnp.sum((1.0 - done_b) * valid)
  out_ref[0, 0] = (term_num / (term_den + 1e-20) * _LEAF_COEF
                   + flow_num / (flow_den + 1e-20))


def _tc_loss(acc, reward, done, interpret=False):
  acc3 = acc.reshape(2, _PAD_T // 128, 128)
  pad = _PAD_T - _T
  rw = jnp.pad(reward, (0, pad)).reshape(_PAD_T // 128, 128)
  dn = jnp.pad(done, (0, pad)).reshape(_PAD_T // 128, 128)
  out = pl.pallas_call(
      _tc_loss_body,
      out_shape=jax.ShapeDtypeStruct((1, 1), jnp.float32),
      out_specs=pl.BlockSpec(memory_space=pltpu.MemorySpace.SMEM),
      interpret=interpret,
  )(acc3, rw, dn)
  return out.reshape(())


def kernel(qsa_p, pb, edge_out_s, seg_s, reward, done):
  acc = _sc_segment_sums(qsa_p, pb.astype(jnp.int32),
                         edge_out_s, seg_s.astype(jnp.int32))
  return _tc_loss(acc, reward, done)
